# Initial kernel scaffold; baseline (speedup 1.0000x reference)
import jax
import jax.numpy as jnp
from jax.experimental import pallas as pl


def kernel(features, edge_index, edge_weight, W1, W2):
    raise NotImplementedError("write your pallas kernel here")


# speedup vs baseline: 5.1463x; 5.1463x over previous
import functools

import jax
import jax.numpy as jnp
from jax import lax
from jax.experimental import pallas as pl
from jax.experimental.pallas import tpu as pltpu
from jax.experimental.pallas import tpu_sc as plsc

N_NODES = 10000
N_EDGES = 320000
D = 128

NC = 2
NS = 16
L = 16

CHUNK = 128
EPW = N_EDGES // (NC * NS)
FULL_CHUNKS = EPW // CHUNK
TAIL = EPW - FULL_CHUNKS * CHUNK

OB = 80
NB = N_NODES // OB
BPS = -(-NB // NS)


def _spmm_body(x_hbm, src_hbm, dst_hbm, w_hbm, part_hbm,
               src_v, dst_v, w_v, rows_v,
               src_t, dst_t, w_t, rows_t,
               zbuf, acc, gsem):
    c = lax.axis_index("c")
    s = lax.axis_index("s")
    wid = c * NS + s

    zero = jnp.zeros((L,), jnp.float32)

    def zfill(i, _):
        for k in range(D // L):
            zbuf[i, pl.ds(k * L, L)] = zero
        return 0

    lax.fori_loop(0, OB, zfill, 0)
    for k in range(BPS):
        b = s + k * NS

        @pl.when(b < NB)
        def _():
            pltpu.sync_copy(zbuf, acc.at[pl.ds(b * OB, OB)])

    plsc.subcore_barrier()

    base0 = wid * EPW

    def do_chunk(base, n, src_b, dst_b, w_b, rows_b):
        pltpu.sync_copy(src_hbm.at[pl.ds(base, n)], src_b)
        pltpu.sync_copy(dst_hbm.at[pl.ds(base, n)], dst_b)
        pltpu.sync_copy(w_hbm.at[pl.ds(base, n)], w_b)
        pltpu.async_copy(x_hbm.at[src_b], rows_b, gsem).wait()

        def mul(g, _):
            wv = w_b[pl.ds(g * L, L)]
            for j in range(L):
                w = wv[j]
                i = g * L + j
                for k in range(D // L):
                    sl = pl.ds(k * L, L)
                    rows_b[i, sl] = rows_b[i, sl] * w
            return 0

        lax.fori_loop(0, n // L, mul, 0)
        pltpu.sync_copy(rows_b, acc.at[dst_b], add=True)

    def chunk_loop(k, _):
        do_chunk(base0 + k * CHUNK, CHUNK, src_v, dst_v, w_v, rows_v)
        return 0

    lax.fori_loop(0, FULL_CHUNKS, chunk_loop, 0)
    if TAIL:
        do_chunk(base0 + FULL_CHUNKS * CHUNK, TAIL, src_t, dst_t, w_t, rows_t)
    plsc.subcore_barrier()

    for k in range(BPS):
        b = s + k * NS

        @pl.when(b < NB)
        def _():
            r = b * OB
            pltpu.sync_copy(acc.at[pl.ds(r, OB)], zbuf)
            pltpu.sync_copy(zbuf, part_hbm.at[c, pl.ds(r, OB)])


_spmm = pl.kernel(
    _spmm_body,
    out_type=jax.ShapeDtypeStruct((NC, N_NODES, D), jnp.float32),
    mesh=plsc.VectorSubcoreMesh(core_axis_name="c", subcore_axis_name="s",
                                num_cores=NC, num_subcores=NS),
    scratch_types=[
        pltpu.VMEM((CHUNK,), jnp.int32),
        pltpu.VMEM((CHUNK,), jnp.int32),
        pltpu.VMEM((CHUNK,), jnp.float32),
        pltpu.VMEM((CHUNK, D), jnp.float32),
        pltpu.VMEM((TAIL,), jnp.int32),
        pltpu.VMEM((TAIL,), jnp.int32),
        pltpu.VMEM((TAIL,), jnp.float32),
        pltpu.VMEM((TAIL, D), jnp.float32),
        pltpu.VMEM((OB, D), jnp.float32),
        pltpu.VMEM_SHARED((N_NODES, D), jnp.float32),
        pltpu.SemaphoreType.DMA,
    ],
)


def _mm_body(x_ref, w_ref, o_ref):
    o_ref[...] = jnp.dot(x_ref[...], w_ref[...],
                         preferred_element_type=jnp.float32)


def _fuse_body(p_ref, w_ref, o_ref):
    h = jnp.maximum(p_ref[0] + p_ref[1], 0.0)
    o_ref[...] = jnp.dot(h, w_ref[...], preferred_element_type=jnp.float32)


def _add_body(q_ref, o_ref):
    o_ref[...] = q_ref[0] + q_ref[1]


_MB = 1000

_mm = pl.pallas_call(
    _mm_body,
    grid=(N_NODES // _MB,),
    in_specs=[pl.BlockSpec((_MB, D), lambda i: (i, 0)),
              pl.BlockSpec((D, D), lambda i: (0, 0))],
    out_specs=pl.BlockSpec((_MB, D), lambda i: (i, 0)),
    out_shape=jax.ShapeDtypeStruct((N_NODES, D), jnp.float32),
)

_fuse = pl.pallas_call(
    _fuse_body,
    grid=(N_NODES // _MB,),
    in_specs=[pl.BlockSpec((NC, _MB, D), lambda i: (0, i, 0)),
              pl.BlockSpec((D, D), lambda i: (0, 0))],
    out_specs=pl.BlockSpec((_MB, D), lambda i: (i, 0)),
    out_shape=jax.ShapeDtypeStruct((N_NODES, D), jnp.float32),
)

_addp = pl.pallas_call(
    _add_body,
    grid=(N_NODES // _MB,),
    in_specs=[pl.BlockSpec((NC, _MB, D), lambda i: (0, i, 0))],
    out_specs=pl.BlockSpec((_MB, D), lambda i: (i, 0)),
    out_shape=jax.ShapeDtypeStruct((N_NODES, D), jnp.float32),
)


@jax.jit
def kernel(features, edge_index, edge_weight, W1, W2):
    src = edge_index[0].astype(jnp.int32)
    dst = edge_index[1].astype(jnp.int32)
    w = edge_weight.astype(jnp.float32)

    support1 = _mm(features, W1)
    p = _spmm(support1, src, dst, w)
    support2 = _fuse(p, W2)
    q = _spmm(support2, src, dst, w)
    return _addp(q)
